# Initial kernel scaffold; baseline (speedup 1.0000x reference)
import jax
import jax.numpy as jnp
from jax.experimental import pallas as pl


def kernel(sentences, in_mu_table, in_cho_table, trans_mu, trans_cho, out_mu, out_cho):
    raise NotImplementedError("write your pallas kernel here")


# speedup vs baseline: 767.7589x; 767.7589x over previous
import functools

import jax
import jax.numpy as jnp
import numpy as np
from jax import lax
from jax.experimental import pallas as pl
from jax.experimental.pallas import tpu as pltpu
from jax.experimental.pallas import tpu_sc as plsc

DIM = 16
BATCH = 32
MAXLEN = 20
NLABELS = 20
LOG2PI = float(np.log(2.0 * np.pi))
PAD_B = 1024


def _sc_gather_body(mu_tab, cho_tab, idx_hbm, out_mu, out_cho,
                    idx_v, rows_mu, rows_cho, sem):
    nc = 2
    wid = lax.axis_index("s") * nc + lax.axis_index("c")
    bpw = PAD_B // 32
    base = wid * bpw
    pltpu.sync_copy(idx_hbm.at[pl.ds(base, bpw)], idx_v)
    pltpu.async_copy(mu_tab.at[idx_v], rows_mu, sem).wait()
    pltpu.async_copy(cho_tab.at[idx_v], rows_cho, sem).wait()
    pltpu.sync_copy(rows_mu, out_mu.at[pl.ds(base, bpw)])
    pltpu.sync_copy(rows_cho, out_cho.at[pl.ds(base, bpw)])


def _sc_gather(mu_tab, cho_tab, idx):
    bpw = PAD_B // 32
    f = functools.partial(
        pl.kernel,
        mesh=plsc.VectorSubcoreMesh(core_axis_name="c", subcore_axis_name="s"),
        out_type=[jax.ShapeDtypeStruct((PAD_B, 32), jnp.float32),
                  jax.ShapeDtypeStruct((PAD_B, 32), jnp.float32)],
        scratch_types=[pltpu.VMEM((bpw,), jnp.int32),
                       pltpu.VMEM((bpw, 32), jnp.float32),
                       pltpu.VMEM((bpw, 32), jnp.float32),
                       pltpu.SemaphoreType.DMA],
        compiler_params=pltpu.CompilerParams(use_tc_tiling_on_sc=False),
    )(_sc_gather_body)
    return f(mu_tab, cho_tab, idx)


def _eye3(n):
    r = lax.broadcasted_iota(jnp.int32, (DIM, DIM, n), 0)
    c = lax.broadcasted_iota(jnp.int32, (DIM, DIM, n), 1)
    return (r == c).astype(jnp.float32)


def _rep2(x):
    parts = []
    for p in range(x.shape[-1] // 32):
        s = x[..., 32 * p:32 * p + 32]
        parts += [s, s]
    return jnp.concatenate(parts, axis=-1)


def _tilel(x, n):
    return jnp.concatenate([x] * n, axis=-1)


def _expand_c(x, reps):
    b0 = jnp.broadcast_to(x[..., 0:1], x.shape[:-1] + (32,))
    b1 = jnp.broadcast_to(x[..., 1:2], x.shape[:-1] + (32,))
    return jnp.concatenate([b0, b1] * reps, axis=-1)


def _gj_solve(C, rhs):
    M = jnp.concatenate([C, rhs], axis=1)
    n = DIM
    logdet = None
    for j in range(n):
        row_j = M[j]
        p = row_j[j]
        lp = jnp.log(p)
        logdet = lp if logdet is None else logdet + lp
        row_jn = row_j / p[None, :]
        coefs = M[:, j:j + 1, :]
        onej = (lax.broadcasted_iota(jnp.int32, (n, 1, 1), 0) == j
                ).astype(jnp.float32)
        M = M - (coefs - onej) * row_jn[None, :, :]
    return M[:, n:, :], logdet


def _bmm(A, B):
    acc = None
    for k in range(DIM):
        t = A[:, k:k + 1, :] * B[k:k + 1, :, :]
        acc = t if acc is None else acc + t
    return acc


def _bmv(A, x):
    return jnp.sum(A * x[None, :, :], axis=1)


def _gmulti_diag(mu0, mu1, var0, dg):
    n = var0.shape[-1]
    eye = _eye3(n)
    C = var0 + eye * dg[None, :, :]
    rhs = jnp.concatenate([mu0[:, None, :], mu1[:, None, :], eye], axis=1)
    sol, logdet = _gj_solve(C, rhs)
    t0, t1, Cinv = sol[:, 0, :], sol[:, 1, :], sol[:, 2:, :]
    diff = mu0 - mu1
    maha = jnp.sum(diff * (t0 - t1), axis=0)
    z = -0.5 * (DIM * LOG2PI + logdet + maha)
    mu = dg * t0 + _bmv(var0, t1)
    var = _bmm(var0, Cinv * dg[None, :, :])
    return z, mu, var


def _gmulti_full(mu0, mu1, var0, var1):
    n = var0.shape[-1]
    C = var0 + var1
    rhs = jnp.concatenate([mu0[:, None, :], mu1[:, None, :], _eye3(n)], axis=1)
    sol, logdet = _gj_solve(C, rhs)
    t0, t1, Cinv = sol[:, 0, :], sol[:, 1, :], sol[:, 2:, :]
    diff = mu0 - mu1
    maha = jnp.sum(diff * (t0 - t1), axis=0)
    z = -0.5 * (DIM * LOG2PI + logdet + maha)
    mu = _bmv(var1, t0) + _bmv(var0, t1)
    var = _bmm(var0, _bmm(Cinv, var1))
    return z, mu, var


def _gintegral(mx, my, Sxx, Sxy, Syx, Syy, mu1, var1):
    C2 = Sxx + var1
    diff = mx - mu1
    rhs = jnp.concatenate([diff[:, None, :], Sxy], axis=1)
    sol, logdet = _gj_solve(C2, rhs)
    t, H = sol[:, 0, :], sol[:, 1:, :]
    maha = jnp.sum(diff * t, axis=0)
    z = -0.5 * (DIM * LOG2PI + logdet + maha)
    mu_y = my - _bmv(Syx, t)
    var_y = Syy - _bmm(Syx, H)
    return z, mu_y, var_y


def _blocks(x, nb):
    return [x[..., 32 * q:32 * q + 32] for q in range(nb)]


def _top2(score, mu, var):
    sb = _blocks(score, 8)
    best1 = sb[0]
    idx1 = jnp.zeros((32,), jnp.float32)
    for q in range(1, 8):
        better = sb[q] > best1
        best1 = jnp.where(better, sb[q], best1)
        idx1 = jnp.where(better, float(q), idx1)
    sb2 = [jnp.where(idx1 == float(q), -1e30, sb[q]) for q in range(8)]
    best2 = sb2[0]
    idx2 = jnp.zeros((32,), jnp.float32)
    for q in range(1, 8):
        better = sb2[q] > best2
        best2 = jnp.where(better, sb2[q], best2)
        idx2 = jnp.where(better, float(q), idx2)
    mb = _blocks(mu, 8)
    vb = _blocks(var, 8)
    pm1 = pm2 = pv1 = pv2 = None
    for q in range(8):
        w1 = (idx1 == float(q)).astype(jnp.float32)[None, :]
        w2 = (idx2 == float(q)).astype(jnp.float32)[None, :]
        a1 = w1 * mb[q]
        a2 = w2 * mb[q]
        b1 = w1[None] * vb[q]
        b2 = w2[None] * vb[q]
        pm1 = a1 if pm1 is None else pm1 + a1
        pm2 = a2 if pm2 is None else pm2 + a2
        pv1 = b1 if pv1 is None else pv1 + b1
        pv2 = b2 if pv2 is None else pv2 + b2
    ps = jnp.concatenate([best1, best2])
    pm = jnp.concatenate([pm1, pm2], axis=-1)
    pv = jnp.concatenate([pv1, pv2], axis=-1)
    return ps, pm, pv


def _tc_body(emu_ref, echo_ref, mx_ref, my_ref, sxx_ref, sxy_ref, syx_ref,
             syy_ref, omu_ref, ocho_ref, out_ref,
             fs_ref, fm_ref, fv_ref, bs_ref, bm_ref, bv_ref):
    B, T, d = BATCH, MAXLEN, DIM

    def run_dir(dirn, s_ref, m_ref, v_ref):
        mx_c = mx_ref[dirn]
        my_c = my_ref[dirn]
        sxx_c = sxx_ref[dirn]
        sxy_c = sxy_ref[dirn]
        syx_c = syx_ref[dirn]
        syy_c = syy_ref[dirn]
        mx64 = _expand_c(mx_c, 1)
        my64 = _expand_c(my_c, 1)
        sxx64 = _expand_c(sxx_c, 1)
        sxy64 = _expand_c(sxy_c, 1)
        syx64 = _expand_c(syx_c, 1)
        syy64 = _expand_c(syy_c, 1)
        z0, m0, v0 = _gintegral(mx64, my64, sxx64, sxy64, syx64, syy64,
                                jnp.zeros((d, 64), jnp.float32), _eye3(64))
        s_c0, s_c1 = z0[0:32], z0[32:64]
        cond = s_c0 >= s_c1
        ps = jnp.concatenate([jnp.where(cond, s_c0, s_c1),
                              jnp.where(cond, s_c1, s_c0)])
        cb = cond[None, :]
        pm = jnp.concatenate([jnp.where(cb, m0[:, 0:32], m0[:, 32:64]),
                              jnp.where(cb, m0[:, 32:64], m0[:, 0:32])], axis=-1)
        cb2 = cond[None, None, :]
        pv = jnp.concatenate([jnp.where(cb2, v0[..., 0:32], v0[..., 32:64]),
                              jnp.where(cb2, v0[..., 32:64], v0[..., 0:32])],
                             axis=-1)

        mx256 = _expand_c(mx_c, 4)
        my256 = _expand_c(my_c, 4)
        sxx256 = _expand_c(sxx_c, 4)
        sxy256 = _expand_c(sxy_c, 4)
        syx256 = _expand_c(syx_c, 4)
        syy256 = _expand_c(syy_c, 4)

        def step(i, carry):
            ps, pm, pv = carry
            s_ref[pl.ds(i, 1)] = ps[None]
            m_ref[pl.ds(i, 1)] = pm[None]
            v_ref[pl.ds(i, 1)] = pv[None]
            ti = i if dirn == 0 else T - 1 - i
            emu_i = emu_ref[pl.ds(ti, 1)][0]
            ed_i = echo_ref[pl.ds(ti, 1)][0] ** 2
            z1, m1, v1 = _gmulti_diag(_rep2(pm), _tilel(emu_i, 2),
                                      _rep2(pv), _tilel(ed_i, 2))
            z2, m2, v2 = _gintegral(mx256, my256, sxx256, sxy256, syx256,
                                    syy256, _rep2(m1), _rep2(v1))
            score = _rep2(_rep2(ps)) + _rep2(z1) + z2
            return _top2(score, m2, v2)

        lax.fori_loop(0, T, step, (ps, pm, pv))

    run_dir(0, fs_ref, fm_ref, fv_ref)
    run_dir(1, bs_ref, bm_ref, bv_ref)

    def pos(i, _):
        fs = fs_ref[pl.ds(i, 1)][0]
        fm = fm_ref[pl.ds(i, 1)][0]
        fv = fv_ref[pl.ds(i, 1)][0]
        bi = T - 1 - i
        bs = bs_ref[pl.ds(bi, 1)][0]
        bm = bm_ref[pl.ds(bi, 1)][0]
        bv = bv_ref[pl.ds(bi, 1)][0]
        emu_i = emu_ref[pl.ds(i, 1)][0]
        ed_i = echo_ref[pl.ds(i, 1)][0] ** 2
        z1, m1, v1 = _gmulti_diag(_rep2(fm), _tilel(emu_i, 2),
                                  _rep2(fv), _tilel(ed_i, 2))
        s1 = _rep2(fs) + z1
        z2, m2, v2 = _gmulti_full(_rep2(m1), _tilel(bm, 4),
                                  _rep2(v1), _tilel(bv, 4))
        s2 = _rep2(s1) + _tilel(bs, 4) + z2

        def lab(l, _):
            od_l = ocho_ref[pl.ds(l, 1)][0] ** 2
            omu_l = omu_ref[pl.ds(l, 1)][0]
            odb = jnp.broadcast_to(od_l, (d, 256))
            omb = jnp.broadcast_to(omu_l, (d, 256))
            C3 = v2 + _eye3(256) * odb[None, :, :]
            diff3 = m2 - omb
            sol, logdet = _gj_solve(C3, diff3[:, None, :])
            maha = jnp.sum(diff3 * sol[:, 0, :], axis=0)
            z3 = -0.5 * (d * LOG2PI + logdet + maha)
            total = s2 + z3
            tb = _blocks(total, 8)
            mt = tb[0]
            for q in range(1, 8):
                mt = jnp.maximum(mt, tb[q])
            acc = None
            for q in range(8):
                e = jnp.exp(tb[q] - mt)
                acc = e if acc is None else acc + e
            lse = jnp.log(acc) + mt
            out_ref[pl.ds(i * NLABELS + l, 1)] = lse[None]
            return 0

        lax.fori_loop(0, NLABELS, lab, 0)
        return 0

    lax.fori_loop(0, T, pos, 0)


def _tc_call(emuT, echoT, mxT, myT, sxxT, sxyT, syxT, syyT, omuT, ochoT):
    scratch = []
    for _ in range(2):
        scratch += [
            pltpu.VMEM((MAXLEN, 64), jnp.float32),
            pltpu.VMEM((MAXLEN, DIM, 64), jnp.float32),
            pltpu.VMEM((MAXLEN, DIM, DIM, 64), jnp.float32),
        ]
    return pl.pallas_call(
        _tc_body,
        out_shape=jax.ShapeDtypeStruct((MAXLEN * NLABELS, BATCH), jnp.float32),
        scratch_shapes=scratch,
    )(emuT, echoT, mxT, myT, sxxT, sxyT, syxT, syyT, omuT, ochoT)


def _post_gather(mu_rows, cho_rows, trans_mu, trans_cho, out_mu, out_cho):
    d = DIM
    emu = mu_rows[:BATCH * MAXLEN].reshape(MAXLEN, BATCH, 2, d)
    echo = cho_rows[:BATCH * MAXLEN].reshape(MAXLEN, BATCH, 2, d)
    emuT = jnp.transpose(emu, (0, 3, 2, 1)).reshape(MAXLEN, d, 64)
    echoT = jnp.transpose(echo, (0, 3, 2, 1)).reshape(MAXLEN, d, 64)
    t_var = trans_cho @ jnp.swapaxes(trans_cho, -1, -2)
    blocks = {
        'xx': (t_var[:, :d, :d], t_var[:, d:, d:]),
        'xy': (t_var[:, :d, d:], t_var[:, d:, :d]),
        'yx': (t_var[:, d:, :d], t_var[:, :d, d:]),
        'yy': (t_var[:, d:, d:], t_var[:, :d, :d]),
    }

    def pack_mat(pair):
        return jnp.stack([jnp.transpose(p, (1, 2, 0)) for p in pair])

    sxxT = pack_mat(blocks['xx'])
    sxyT = pack_mat(blocks['xy'])
    syxT = pack_mat(blocks['yx'])
    syyT = pack_mat(blocks['yy'])
    mxT = jnp.stack([jnp.transpose(trans_mu[:, :d]),
                     jnp.transpose(trans_mu[:, d:])])
    myT = jnp.stack([jnp.transpose(trans_mu[:, d:]),
                     jnp.transpose(trans_mu[:, :d])])
    omuT = out_mu.reshape(NLABELS, d)[:, :, None]
    ochoT = out_cho.reshape(NLABELS, d)[:, :, None]
    out = _tc_call(emuT, echoT, mxT, myT, sxxT, sxyT, syxT, syyT, omuT, ochoT)
    return jnp.transpose(out.reshape(MAXLEN, NLABELS, BATCH), (2, 0, 1))


def kernel(sentences, in_mu_table, in_cho_table, trans_mu, trans_cho,
           out_mu, out_cho):
    idx = jnp.concatenate([
        jnp.transpose(sentences).reshape(-1).astype(jnp.int32),
        jnp.zeros((PAD_B - BATCH * MAXLEN,), jnp.int32),
    ])
    mu_rows, cho_rows = _sc_gather(in_mu_table, in_cho_table, idx)
    return _post_gather(mu_rows, cho_rows, trans_mu, trans_cho, out_mu, out_cho)


# speedup vs baseline: 1123.9079x; 1.4639x over previous
import functools

import jax
import jax.numpy as jnp
import numpy as np
from jax import lax
from jax.experimental import pallas as pl
from jax.experimental.pallas import tpu as pltpu
from jax.experimental.pallas import tpu_sc as plsc

DIM = 16
BATCH = 32
MAXLEN = 20
NLABELS = 20
LOG2PI = float(np.log(2.0 * np.pi))
PAD_B = 1024


def _sc_gather_body(mu_tab, cho_tab, idx_hbm, out_mu, out_cho,
                    idx_v, rows_mu, rows_cho, sem):
    nc = 2
    wid = lax.axis_index("s") * nc + lax.axis_index("c")
    bpw = PAD_B // 32
    base = wid * bpw
    pltpu.sync_copy(idx_hbm.at[pl.ds(base, bpw)], idx_v)
    pltpu.async_copy(mu_tab.at[idx_v], rows_mu, sem).wait()
    pltpu.async_copy(cho_tab.at[idx_v], rows_cho, sem).wait()
    pltpu.sync_copy(rows_mu, out_mu.at[pl.ds(base, bpw)])
    pltpu.sync_copy(rows_cho, out_cho.at[pl.ds(base, bpw)])


def _sc_gather(mu_tab, cho_tab, idx):
    bpw = PAD_B // 32
    f = functools.partial(
        pl.kernel,
        mesh=plsc.VectorSubcoreMesh(core_axis_name="c", subcore_axis_name="s"),
        out_type=[jax.ShapeDtypeStruct((PAD_B, 32), jnp.float32),
                  jax.ShapeDtypeStruct((PAD_B, 32), jnp.float32)],
        scratch_types=[pltpu.VMEM((bpw,), jnp.int32),
                       pltpu.VMEM((bpw, 32), jnp.float32),
                       pltpu.VMEM((bpw, 32), jnp.float32),
                       pltpu.SemaphoreType.DMA],
        compiler_params=pltpu.CompilerParams(use_tc_tiling_on_sc=False),
    )(_sc_gather_body)
    return f(mu_tab, cho_tab, idx)


def _eye3(n):
    r = lax.broadcasted_iota(jnp.int32, (DIM, DIM, n), 0)
    c = lax.broadcasted_iota(jnp.int32, (DIM, DIM, n), 1)
    return (r == c).astype(jnp.float32)


def _rep2(x):
    parts = []
    for p in range(x.shape[-1] // 32):
        s = x[..., 32 * p:32 * p + 32]
        parts += [s, s]
    return jnp.concatenate(parts, axis=-1)


def _tilel(x, n):
    return jnp.concatenate([x] * n, axis=-1)


def _expand_c(x, reps):
    b0 = jnp.broadcast_to(x[..., 0:1], x.shape[:-1] + (32,))
    b1 = jnp.broadcast_to(x[..., 1:2], x.shape[:-1] + (32,))
    return jnp.concatenate([b0, b1] * reps, axis=-1)


def _gj_solve(C, rhs):
    M = jnp.concatenate([C, rhs], axis=0)
    n = DIM
    logdet = None
    for j in range(n):
        row_j = M[:, j, :]
        p = row_j[j]
        lp = jnp.log(p)
        logdet = lp if logdet is None else logdet + lp
        row_jn = row_j * (1.0 / p)[None, :]
        coefs = M[j]
        onej = (lax.broadcasted_iota(jnp.int32, (1, n, 1), 1) == j
                ).astype(jnp.float32)
        M = M - (coefs[None, :, :] - onej) * row_jn[:, None, :]
    return M[n:], logdet


def _sym_z(C, diff):
    S = jnp.concatenate([C, diff[None]], axis=0)
    logdet = None
    maha = None
    for j in range(DIM):
        p = S[0, 0]
        lp = jnp.log(p)
        logdet = lp if logdet is None else logdet + lp
        bj = S[-1, 0]
        m = bj * bj * (1.0 / p)
        maha = m if maha is None else maha + m
        if j < DIM - 1:
            col0 = S[1:, 0, :]
            mult = S[0, 1:, :] * (1.0 / p)[None, :]
            S = S[1:, 1:, :] - col0[:, None, :] * mult[None, :, :]
    return logdet, maha


def _bmmT(A, H):
    acc = None
    for r in range(DIM):
        t = A[:, r:r + 1, :] * H[:, r, :][None, :, :]
        acc = t if acc is None else acc + t
    return acc


def _bmv(A, x):
    return jnp.sum(A * x[None, :, :], axis=1)


def _gmulti_diag(mu0, mu1, var0, dg):
    n = var0.shape[-1]
    eye = _eye3(n)
    C = var0 + eye * dg[None, :, :]
    rhs = jnp.concatenate([mu0[None], mu1[None], eye], axis=0)
    sol, logdet = _gj_solve(C, rhs)
    t0, t1, Cinv = sol[0], sol[1], sol[2:]
    diff = mu0 - mu1
    maha = jnp.sum(diff * (t0 - t1), axis=0)
    z = -0.5 * (DIM * LOG2PI + logdet + maha)
    mu = mu1 + dg * (t0 - t1)
    var = eye * dg[:, None, :] - Cinv * dg[:, None, :] * dg[None, :, :]
    return z, mu, var


def _gmulti_full(mu0, mu1, var0, var1):
    C = var0 + var1
    rhs = jnp.concatenate([mu0[None], mu1[None], var1], axis=0)
    sol, logdet = _gj_solve(C, rhs)
    t0, t1, H = sol[0], sol[1], sol[2:]
    diff = mu0 - mu1
    maha = jnp.sum(diff * (t0 - t1), axis=0)
    z = -0.5 * (DIM * LOG2PI + logdet + maha)
    mu = mu1 + _bmv(var1, t0 - t1)
    var = var1 - _bmmT(var1, H)
    return z, mu, var


def _gintegral(mx, my, Sxx, Sxy, Syx, Syy, mu1, var1):
    C2 = Sxx + var1
    diff = mx - mu1
    rhs = jnp.concatenate([diff[None], Syx], axis=0)
    sol, logdet = _gj_solve(C2, rhs)
    t, H = sol[0], sol[1:]
    maha = jnp.sum(diff * t, axis=0)
    z = -0.5 * (DIM * LOG2PI + logdet + maha)
    mu_y = my - _bmv(Syx, t)
    var_y = Syy - _bmmT(Syx, H)
    return z, mu_y, var_y


def _blocks(x, nb):
    return [x[..., 32 * q:32 * q + 32] for q in range(nb)]


def _top2(score, mu, var):
    sb = _blocks(score, 8)
    best1 = sb[0]
    idx1 = jnp.zeros((32,), jnp.float32)
    for q in range(1, 8):
        better = sb[q] > best1
        best1 = jnp.where(better, sb[q], best1)
        idx1 = jnp.where(better, float(q), idx1)
    sb2 = [jnp.where(idx1 == float(q), -1e30, sb[q]) for q in range(8)]
    best2 = sb2[0]
    idx2 = jnp.zeros((32,), jnp.float32)
    for q in range(1, 8):
        better = sb2[q] > best2
        best2 = jnp.where(better, sb2[q], best2)
        idx2 = jnp.where(better, float(q), idx2)
    mb = _blocks(mu, 8)
    vb = _blocks(var, 8)
    pm1 = pm2 = pv1 = pv2 = None
    for q in range(8):
        w1 = (idx1 == float(q)).astype(jnp.float32)[None, :]
        w2 = (idx2 == float(q)).astype(jnp.float32)[None, :]
        a1 = w1 * mb[q]
        a2 = w2 * mb[q]
        b1 = w1[None] * vb[q]
        b2 = w2[None] * vb[q]
        pm1 = a1 if pm1 is None else pm1 + a1
        pm2 = a2 if pm2 is None else pm2 + a2
        pv1 = b1 if pv1 is None else pv1 + b1
        pv2 = b2 if pv2 is None else pv2 + b2
    ps = jnp.concatenate([best1, best2])
    pm = jnp.concatenate([pm1, pm2], axis=-1)
    pv = jnp.concatenate([pv1, pv2], axis=-1)
    return ps, pm, pv


def _tc_body(emu_ref, echo_ref, mx_ref, my_ref, sxx_ref, sxy_ref, syx_ref,
             syy_ref, omu_ref, ocho_ref, out_ref,
             fs_ref, fm_ref, fv_ref, bs_ref, bm_ref, bv_ref):
    B, T, d = BATCH, MAXLEN, DIM

    def run_dir(dirn, s_ref, m_ref, v_ref):
        mx_c = mx_ref[dirn]
        my_c = my_ref[dirn]
        sxx_c = sxx_ref[dirn]
        sxy_c = sxy_ref[dirn]
        syx_c = syx_ref[dirn]
        syy_c = syy_ref[dirn]
        mx64 = _expand_c(mx_c, 1)
        my64 = _expand_c(my_c, 1)
        sxx64 = _expand_c(sxx_c, 1)
        sxy64 = _expand_c(sxy_c, 1)
        syx64 = _expand_c(syx_c, 1)
        syy64 = _expand_c(syy_c, 1)
        z0, m0, v0 = _gintegral(mx64, my64, sxx64, sxy64, syx64, syy64,
                                jnp.zeros((d, 64), jnp.float32), _eye3(64))
        s_c0, s_c1 = z0[0:32], z0[32:64]
        cond = s_c0 >= s_c1
        ps = jnp.concatenate([jnp.where(cond, s_c0, s_c1),
                              jnp.where(cond, s_c1, s_c0)])
        cb = cond[None, :]
        pm = jnp.concatenate([jnp.where(cb, m0[:, 0:32], m0[:, 32:64]),
                              jnp.where(cb, m0[:, 32:64], m0[:, 0:32])], axis=-1)
        cb2 = cond[None, None, :]
        pv = jnp.concatenate([jnp.where(cb2, v0[..., 0:32], v0[..., 32:64]),
                              jnp.where(cb2, v0[..., 32:64], v0[..., 0:32])],
                             axis=-1)

        mx256 = _expand_c(mx_c, 4)
        my256 = _expand_c(my_c, 4)
        sxx256 = _expand_c(sxx_c, 4)
        sxy256 = _expand_c(sxy_c, 4)
        syx256 = _expand_c(syx_c, 4)
        syy256 = _expand_c(syy_c, 4)

        def step(i, carry):
            ps, pm, pv = carry
            s_ref[pl.ds(i, 1)] = ps[None]
            m_ref[pl.ds(i, 1)] = pm[None]
            v_ref[pl.ds(i, 1)] = pv[None]
            ti = i if dirn == 0 else T - 1 - i
            emu_i = emu_ref[pl.ds(ti, 1)][0]
            ed_i = echo_ref[pl.ds(ti, 1)][0] ** 2
            z1, m1, v1 = _gmulti_diag(_rep2(pm), _tilel(emu_i, 2),
                                      _rep2(pv), _tilel(ed_i, 2))
            z2, m2, v2 = _gintegral(mx256, my256, sxx256, sxy256, syx256,
                                    syy256, _rep2(m1), _rep2(v1))
            score = _rep2(_rep2(ps)) + _rep2(z1) + z2
            return _top2(score, m2, v2)

        lax.fori_loop(0, T, step, (ps, pm, pv))

    run_dir(0, fs_ref, fm_ref, fv_ref)
    run_dir(1, bs_ref, bm_ref, bv_ref)

    def pos(i, _):
        fs = fs_ref[pl.ds(i, 1)][0]
        fm = fm_ref[pl.ds(i, 1)][0]
        fv = fv_ref[pl.ds(i, 1)][0]
        bi = T - 1 - i
        bs = bs_ref[pl.ds(bi, 1)][0]
        bm = bm_ref[pl.ds(bi, 1)][0]
        bv = bv_ref[pl.ds(bi, 1)][0]
        emu_i = emu_ref[pl.ds(i, 1)][0]
        ed_i = echo_ref[pl.ds(i, 1)][0] ** 2
        z1, m1, v1 = _gmulti_diag(_rep2(fm), _tilel(emu_i, 2),
                                  _rep2(fv), _tilel(ed_i, 2))
        s1 = _rep2(fs) + z1
        z2, m2, v2 = _gmulti_full(_rep2(m1), _tilel(bm, 4),
                                  _rep2(v1), _tilel(bv, 4))
        s2 = _rep2(s1) + _tilel(bs, 4) + z2

        def lab(l, _):
            od_l = ocho_ref[pl.ds(l, 1)][0] ** 2
            omu_l = omu_ref[pl.ds(l, 1)][0]
            odb = jnp.broadcast_to(od_l, (d, 256))
            omb = jnp.broadcast_to(omu_l, (d, 256))
            C3 = v2 + _eye3(256) * odb[None, :, :]
            diff3 = m2 - omb
            logdet, maha = _sym_z(C3, diff3)
            z3 = -0.5 * (d * LOG2PI + logdet + maha)
            total = s2 + z3
            tb = _blocks(total, 8)
            mt = tb[0]
            for q in range(1, 8):
                mt = jnp.maximum(mt, tb[q])
            acc = None
            for q in range(8):
                e = jnp.exp(tb[q] - mt)
                acc = e if acc is None else acc + e
            lse = jnp.log(acc) + mt
            out_ref[pl.ds(i * NLABELS + l, 1)] = lse[None]
            return 0

        lax.fori_loop(0, NLABELS, lab, 0)
        return 0

    lax.fori_loop(0, T, pos, 0)


def _tc_call(emuT, echoT, mxT, myT, sxxT, sxyT, syxT, syyT, omuT, ochoT):
    scratch = []
    for _ in range(2):
        scratch += [
            pltpu.VMEM((MAXLEN, 64), jnp.float32),
            pltpu.VMEM((MAXLEN, DIM, 64), jnp.float32),
            pltpu.VMEM((MAXLEN, DIM, DIM, 64), jnp.float32),
        ]
    return pl.pallas_call(
        _tc_body,
        out_shape=jax.ShapeDtypeStruct((MAXLEN * NLABELS, BATCH), jnp.float32),
        scratch_shapes=scratch,
    )(emuT, echoT, mxT, myT, sxxT, sxyT, syxT, syyT, omuT, ochoT)


def _post_gather(mu_rows, cho_rows, trans_mu, trans_cho, out_mu, out_cho):
    d = DIM
    emu = mu_rows[:BATCH * MAXLEN].reshape(MAXLEN, BATCH, 2, d)
    echo = cho_rows[:BATCH * MAXLEN].reshape(MAXLEN, BATCH, 2, d)
    emuT = jnp.transpose(emu, (0, 3, 2, 1)).reshape(MAXLEN, d, 64)
    echoT = jnp.transpose(echo, (0, 3, 2, 1)).reshape(MAXLEN, d, 64)
    t_var = trans_cho @ jnp.swapaxes(trans_cho, -1, -2)
    blocks = {
        'xx': (t_var[:, :d, :d], t_var[:, d:, d:]),
        'xy': (t_var[:, :d, d:], t_var[:, d:, :d]),
        'yx': (t_var[:, d:, :d], t_var[:, :d, d:]),
        'yy': (t_var[:, d:, d:], t_var[:, :d, :d]),
    }

    def pack_mat(pair):
        return jnp.stack([jnp.transpose(p, (1, 2, 0)) for p in pair])

    sxxT = pack_mat(blocks['xx'])
    sxyT = pack_mat(blocks['xy'])
    syxT = pack_mat(blocks['yx'])
    syyT = pack_mat(blocks['yy'])
    mxT = jnp.stack([jnp.transpose(trans_mu[:, :d]),
                     jnp.transpose(trans_mu[:, d:])])
    myT = jnp.stack([jnp.transpose(trans_mu[:, d:]),
                     jnp.transpose(trans_mu[:, :d])])
    omuT = out_mu.reshape(NLABELS, d)[:, :, None]
    ochoT = out_cho.reshape(NLABELS, d)[:, :, None]
    out = _tc_call(emuT, echoT, mxT, myT, sxxT, sxyT, syxT, syyT, omuT, ochoT)
    return jnp.transpose(out.reshape(MAXLEN, NLABELS, BATCH), (2, 0, 1))


def kernel(sentences, in_mu_table, in_cho_table, trans_mu, trans_cho,
           out_mu, out_cho):
    idx = jnp.concatenate([
        jnp.transpose(sentences).reshape(-1).astype(jnp.int32),
        jnp.zeros((PAD_B - BATCH * MAXLEN,), jnp.int32),
    ])
    mu_rows, cho_rows = _sc_gather(in_mu_table, in_cho_table, idx)
    return _post_gather(mu_rows, cho_rows, trans_mu, trans_cho, out_mu, out_cho)


# speedup vs baseline: 1257.5591x; 1.1189x over previous
import functools

import jax
import jax.numpy as jnp
import numpy as np
from jax import lax
from jax.experimental import pallas as pl
from jax.experimental.pallas import tpu as pltpu
from jax.experimental.pallas import tpu_sc as plsc

DIM = 16
BATCH = 32
MAXLEN = 20
NLABELS = 20
LOG2PI = float(np.log(2.0 * np.pi))
PAD_B = 1024


def _sc_gather_body(mu_tab, cho_tab, idx_hbm, out_mu, out_cho,
                    idx_v, rows_mu, rows_cho, sem):
    nc = 2
    wid = lax.axis_index("s") * nc + lax.axis_index("c")
    bpw = PAD_B // 32
    base = wid * bpw
    pltpu.sync_copy(idx_hbm.at[pl.ds(base, bpw)], idx_v)
    pltpu.async_copy(mu_tab.at[idx_v], rows_mu, sem).wait()
    pltpu.async_copy(cho_tab.at[idx_v], rows_cho, sem).wait()
    pltpu.sync_copy(rows_mu, out_mu.at[pl.ds(base, bpw)])
    pltpu.sync_copy(rows_cho, out_cho.at[pl.ds(base, bpw)])


def _sc_gather(mu_tab, cho_tab, idx):
    bpw = PAD_B // 32
    f = functools.partial(
        pl.kernel,
        mesh=plsc.VectorSubcoreMesh(core_axis_name="c", subcore_axis_name="s"),
        out_type=[jax.ShapeDtypeStruct((PAD_B, 32), jnp.float32),
                  jax.ShapeDtypeStruct((PAD_B, 32), jnp.float32)],
        scratch_types=[pltpu.VMEM((bpw,), jnp.int32),
                       pltpu.VMEM((bpw, 32), jnp.float32),
                       pltpu.VMEM((bpw, 32), jnp.float32),
                       pltpu.SemaphoreType.DMA],
        compiler_params=pltpu.CompilerParams(use_tc_tiling_on_sc=False),
    )(_sc_gather_body)
    return f(mu_tab, cho_tab, idx)


def _eye3(n):
    r = lax.broadcasted_iota(jnp.int32, (DIM, DIM, n), 0)
    c = lax.broadcasted_iota(jnp.int32, (DIM, DIM, n), 1)
    return (r == c).astype(jnp.float32)


def _rep2(x):
    parts = []
    for p in range(x.shape[-1] // 32):
        s = x[..., 32 * p:32 * p + 32]
        parts += [s, s]
    return jnp.concatenate(parts, axis=-1)


def _tilel(x, n):
    return jnp.concatenate([x] * n, axis=-1)


def _expand_c(x, reps):
    b0 = jnp.broadcast_to(x[..., 0:1], x.shape[:-1] + (32,))
    b1 = jnp.broadcast_to(x[..., 1:2], x.shape[:-1] + (32,))
    return jnp.concatenate([b0, b1] * reps, axis=-1)


def _gj_solve(C, rhs):
    M = jnp.concatenate([C, rhs], axis=0)
    n = DIM
    logdet = None
    for j in range(n):
        row_j = M[:, j, :]
        p = row_j[j]
        lp = jnp.log(p)
        logdet = lp if logdet is None else logdet + lp
        row_jn = row_j * (1.0 / p)[None, :]
        coefs = M[j]
        onej = (lax.broadcasted_iota(jnp.int32, (1, n, 1), 1) == j
                ).astype(jnp.float32)
        M = M - (coefs[None, :, :] - onej) * row_jn[:, None, :]
    return M[n:], logdet


def _sym_z(C, diff):
    S = jnp.concatenate([C, diff[None]], axis=0)
    logdet = None
    maha = None
    for j in range(DIM):
        p = S[0, 0]
        lp = jnp.log(p)
        logdet = lp if logdet is None else logdet + lp
        bj = S[-1, 0]
        m = bj * bj * (1.0 / p)
        maha = m if maha is None else maha + m
        if j < DIM - 1:
            col0 = S[1:, 0, :]
            mult = S[0, 1:, :] * (1.0 / p)[None, :]
            S = S[1:, 1:, :] - col0[:, None, :] * mult[None, :, :]
    return logdet, maha


def _bmmT(A, H):
    acc = None
    for r in range(DIM):
        t = A[:, r:r + 1, :] * H[:, r, :][None, :, :]
        acc = t if acc is None else acc + t
    return acc


def _bmv(A, x):
    return jnp.sum(A * x[None, :, :], axis=1)


def _gmulti_diag(mu0, mu1, var0, dg):
    n = var0.shape[-1]
    eye = _eye3(n)
    C = var0 + eye * dg[None, :, :]
    rhs = jnp.concatenate([mu0[None], mu1[None], eye], axis=0)
    sol, logdet = _gj_solve(C, rhs)
    t0, t1, Cinv = sol[0], sol[1], sol[2:]
    diff = mu0 - mu1
    maha = jnp.sum(diff * (t0 - t1), axis=0)
    z = -0.5 * (DIM * LOG2PI + logdet + maha)
    mu = mu1 + dg * (t0 - t1)
    var = eye * dg[:, None, :] - Cinv * dg[:, None, :] * dg[None, :, :]
    return z, mu, var


def _gmulti_full(mu0, mu1, var0, var1):
    C = var0 + var1
    rhs = jnp.concatenate([mu0[None], mu1[None], var1], axis=0)
    sol, logdet = _gj_solve(C, rhs)
    t0, t1, H = sol[0], sol[1], sol[2:]
    diff = mu0 - mu1
    maha = jnp.sum(diff * (t0 - t1), axis=0)
    z = -0.5 * (DIM * LOG2PI + logdet + maha)
    mu = mu1 + _bmv(var1, t0 - t1)
    var = var1 - _bmmT(var1, H)
    return z, mu, var


def _gintegral(mx, my, Sxx, Sxy, Syx, Syy, mu1, var1):
    C2 = Sxx + var1
    diff = mx - mu1
    rhs = jnp.concatenate([diff[None], Syx], axis=0)
    sol, logdet = _gj_solve(C2, rhs)
    t, H = sol[0], sol[1:]
    maha = jnp.sum(diff * t, axis=0)
    z = -0.5 * (DIM * LOG2PI + logdet + maha)
    mu_y = my - _bmv(Syx, t)
    var_y = Syy - _bmmT(Syx, H)
    return z, mu_y, var_y


def _blocks(x, nb):
    return [x[..., 32 * q:32 * q + 32] for q in range(nb)]


def _top2(score, mu, var):
    sb = _blocks(score, 8)
    best1 = sb[0]
    idx1 = jnp.zeros((32,), jnp.float32)
    for q in range(1, 8):
        better = sb[q] > best1
        best1 = jnp.where(better, sb[q], best1)
        idx1 = jnp.where(better, float(q), idx1)
    sb2 = [jnp.where(idx1 == float(q), -1e30, sb[q]) for q in range(8)]
    best2 = sb2[0]
    idx2 = jnp.zeros((32,), jnp.float32)
    for q in range(1, 8):
        better = sb2[q] > best2
        best2 = jnp.where(better, sb2[q], best2)
        idx2 = jnp.where(better, float(q), idx2)
    mb = _blocks(mu, 8)
    vb = _blocks(var, 8)
    pm1 = pm2 = pv1 = pv2 = None
    for q in range(8):
        w1 = (idx1 == float(q)).astype(jnp.float32)[None, :]
        w2 = (idx2 == float(q)).astype(jnp.float32)[None, :]
        a1 = w1 * mb[q]
        a2 = w2 * mb[q]
        b1 = w1[None] * vb[q]
        b2 = w2[None] * vb[q]
        pm1 = a1 if pm1 is None else pm1 + a1
        pm2 = a2 if pm2 is None else pm2 + a2
        pv1 = b1 if pv1 is None else pv1 + b1
        pv2 = b2 if pv2 is None else pv2 + b2
    ps = jnp.concatenate([best1, best2])
    pm = jnp.concatenate([pm1, pm2], axis=-1)
    pv = jnp.concatenate([pv1, pv2], axis=-1)
    return ps, pm, pv


def _tc_body(emu_ref, echo_ref, mx_ref, my_ref, sxx_ref, sxy_ref, syx_ref,
             syy_ref, omu_ref, ocho_ref, out_ref,
             fs_ref, fm_ref, fv_ref, bs_ref, bm_ref, bv_ref):
    B, T, d = BATCH, MAXLEN, DIM

    def run_dir(dirn, s_ref, m_ref, v_ref):
        mx_c = mx_ref[dirn]
        my_c = my_ref[dirn]
        sxx_c = sxx_ref[dirn]
        sxy_c = sxy_ref[dirn]
        syx_c = syx_ref[dirn]
        syy_c = syy_ref[dirn]
        mx64 = _expand_c(mx_c, 1)
        my64 = _expand_c(my_c, 1)
        sxx64 = _expand_c(sxx_c, 1)
        sxy64 = _expand_c(sxy_c, 1)
        syx64 = _expand_c(syx_c, 1)
        syy64 = _expand_c(syy_c, 1)
        z0, m0, v0 = _gintegral(mx64, my64, sxx64, sxy64, syx64, syy64,
                                jnp.zeros((d, 64), jnp.float32), _eye3(64))
        s_c0, s_c1 = z0[0:32], z0[32:64]
        cond = s_c0 >= s_c1
        ps = jnp.concatenate([jnp.where(cond, s_c0, s_c1),
                              jnp.where(cond, s_c1, s_c0)])
        cb = cond[None, :]
        pm = jnp.concatenate([jnp.where(cb, m0[:, 0:32], m0[:, 32:64]),
                              jnp.where(cb, m0[:, 32:64], m0[:, 0:32])], axis=-1)
        cb2 = cond[None, None, :]
        pv = jnp.concatenate([jnp.where(cb2, v0[..., 0:32], v0[..., 32:64]),
                              jnp.where(cb2, v0[..., 32:64], v0[..., 0:32])],
                             axis=-1)

        mx256 = _expand_c(mx_c, 4)
        my256 = _expand_c(my_c, 4)
        sxx256 = _expand_c(sxx_c, 4)
        sxy256 = _expand_c(sxy_c, 4)
        syx256 = _expand_c(syx_c, 4)
        syy256 = _expand_c(syy_c, 4)

        def step(i, carry):
            ps, pm, pv = carry
            s_ref[pl.ds(i, 1)] = ps[None]
            m_ref[pl.ds(i, 1)] = pm[None]
            v_ref[pl.ds(i, 1)] = pv[None]
            ti = i if dirn == 0 else T - 1 - i
            emu_i = emu_ref[pl.ds(ti, 1)][0]
            ed_i = echo_ref[pl.ds(ti, 1)][0] ** 2
            z1, m1, v1 = _gmulti_diag(_rep2(pm), _tilel(emu_i, 2),
                                      _rep2(pv), _tilel(ed_i, 2))
            z2, m2, v2 = _gintegral(mx256, my256, sxx256, sxy256, syx256,
                                    syy256, _rep2(m1), _rep2(v1))
            score = _rep2(_rep2(ps)) + _rep2(z1) + z2
            return _top2(score, m2, v2)

        lax.fori_loop(0, T, step, (ps, pm, pv))

    run_dir(0, fs_ref, fm_ref, fv_ref)
    run_dir(1, bs_ref, bm_ref, bv_ref)

    NL = NLABELS
    LW = NL * 256
    eyeL = _eye3(LW)
    od_parts = []
    om_parts = []
    for l in range(NL):
        od_l = ocho_ref[pl.ds(l, 1)][0] ** 2
        omu_l = omu_ref[pl.ds(l, 1)][0]
        od_parts.append(jnp.broadcast_to(od_l, (d, 256)))
        om_parts.append(jnp.broadcast_to(omu_l, (d, 256)))
    odb = jnp.concatenate(od_parts, axis=-1)
    omb = jnp.concatenate(om_parts, axis=-1)
    dmat = eyeL * odb[None, :, :]

    def pos(i, _):
        fs = fs_ref[pl.ds(i, 1)][0]
        fm = fm_ref[pl.ds(i, 1)][0]
        fv = fv_ref[pl.ds(i, 1)][0]
        bi = T - 1 - i
        bs = bs_ref[pl.ds(bi, 1)][0]
        bm = bm_ref[pl.ds(bi, 1)][0]
        bv = bv_ref[pl.ds(bi, 1)][0]
        emu_i = emu_ref[pl.ds(i, 1)][0]
        ed_i = echo_ref[pl.ds(i, 1)][0] ** 2
        z1, m1, v1 = _gmulti_diag(_rep2(fm), _tilel(emu_i, 2),
                                  _rep2(fv), _tilel(ed_i, 2))
        s1 = _rep2(fs) + z1
        z2, m2, v2 = _gmulti_full(_rep2(m1), _tilel(bm, 4),
                                  _rep2(v1), _tilel(bv, 4))
        s2 = _rep2(s1) + _tilel(bs, 4) + z2

        C3 = _tilel(v2, NL) + dmat
        diff3 = _tilel(m2, NL) - omb
        logdet, maha = _sym_z(C3, diff3)
        z3 = -0.5 * (d * LOG2PI + logdet + maha)
        total = _tilel(s2, NL) + z3
        for l in range(NL):
            tb = [total[l * 256 + q * 32:l * 256 + q * 32 + 32]
                  for q in range(8)]
            mt = tb[0]
            for q in range(1, 8):
                mt = jnp.maximum(mt, tb[q])
            acc = None
            for q in range(8):
                e = jnp.exp(tb[q] - mt)
                acc = e if acc is None else acc + e
            lse = jnp.log(acc) + mt
            out_ref[pl.ds(i * NLABELS + l, 1)] = lse[None]
        return 0

    lax.fori_loop(0, T, pos, 0)


def _tc_call(emuT, echoT, mxT, myT, sxxT, sxyT, syxT, syyT, omuT, ochoT):
    scratch = []
    for _ in range(2):
        scratch += [
            pltpu.VMEM((MAXLEN, 64), jnp.float32),
            pltpu.VMEM((MAXLEN, DIM, 64), jnp.float32),
            pltpu.VMEM((MAXLEN, DIM, DIM, 64), jnp.float32),
        ]
    return pl.pallas_call(
        _tc_body,
        out_shape=jax.ShapeDtypeStruct((MAXLEN * NLABELS, BATCH), jnp.float32),
        scratch_shapes=scratch,
    )(emuT, echoT, mxT, myT, sxxT, sxyT, syxT, syyT, omuT, ochoT)


def _post_gather(mu_rows, cho_rows, trans_mu, trans_cho, out_mu, out_cho):
    d = DIM
    emu = mu_rows[:BATCH * MAXLEN].reshape(MAXLEN, BATCH, 2, d)
    echo = cho_rows[:BATCH * MAXLEN].reshape(MAXLEN, BATCH, 2, d)
    emuT = jnp.transpose(emu, (0, 3, 2, 1)).reshape(MAXLEN, d, 64)
    echoT = jnp.transpose(echo, (0, 3, 2, 1)).reshape(MAXLEN, d, 64)
    t_var = trans_cho @ jnp.swapaxes(trans_cho, -1, -2)
    blocks = {
        'xx': (t_var[:, :d, :d], t_var[:, d:, d:]),
        'xy': (t_var[:, :d, d:], t_var[:, d:, :d]),
        'yx': (t_var[:, d:, :d], t_var[:, :d, d:]),
        'yy': (t_var[:, d:, d:], t_var[:, :d, :d]),
    }

    def pack_mat(pair):
        return jnp.stack([jnp.transpose(p, (1, 2, 0)) for p in pair])

    sxxT = pack_mat(blocks['xx'])
    sxyT = pack_mat(blocks['xy'])
    syxT = pack_mat(blocks['yx'])
    syyT = pack_mat(blocks['yy'])
    mxT = jnp.stack([jnp.transpose(trans_mu[:, :d]),
                     jnp.transpose(trans_mu[:, d:])])
    myT = jnp.stack([jnp.transpose(trans_mu[:, d:]),
                     jnp.transpose(trans_mu[:, :d])])
    omuT = out_mu.reshape(NLABELS, d)[:, :, None]
    ochoT = out_cho.reshape(NLABELS, d)[:, :, None]
    out = _tc_call(emuT, echoT, mxT, myT, sxxT, sxyT, syxT, syyT, omuT, ochoT)
    return jnp.transpose(out.reshape(MAXLEN, NLABELS, BATCH), (2, 0, 1))


def kernel(sentences, in_mu_table, in_cho_table, trans_mu, trans_cho,
           out_mu, out_cho):
    idx = jnp.concatenate([
        jnp.transpose(sentences).reshape(-1).astype(jnp.int32),
        jnp.zeros((PAD_B - BATCH * MAXLEN,), jnp.int32),
    ])
    mu_rows, cho_rows = _sc_gather(in_mu_table, in_cho_table, idx)
    return _post_gather(mu_rows, cho_rows, trans_mu, trans_cho, out_mu, out_cho)
